# baseline (speedup 1.0000x reference)
import functools

import jax
import jax.numpy as jnp
from jax import lax
from jax.experimental import pallas as pl
from jax.experimental.pallas import tpu as pltpu
from jax.experimental.pallas import tpu_sc as plsc

_D = 2048
_K = 36
_KP = 48
_JITTER = 0.02
_NC = 2
_NS = 16
_NW = _NC * _NS
_L = 16
_CHUNK = 16


def _stats_body(n_ref, gate_ref, sum_ref, g_ref):
    i = pl.program_id(0)
    s = jnp.sum(n_ref[...], axis=0, keepdims=True)

    @pl.when(i == 0)
    def _init():
        sum_ref[...] = s
        g_ref[...] = jax.nn.sigmoid(gate_ref[...])

    @pl.when(i != 0)
    def _acc():
        sum_ref[...] += s


def _noise_stats(noise2d, gate11, rows):
    blk = 1024
    grid = rows // blk
    return pl.pallas_call(
        _stats_body,
        grid=(grid,),
        in_specs=[
            pl.BlockSpec((blk, _K), lambda i: (i, 0)),
            pl.BlockSpec((1, 1), lambda i: (0, 0)),
        ],
        out_specs=[
            pl.BlockSpec((1, _K), lambda i: (0, 0)),
            pl.BlockSpec((1, 1), lambda i: (0, 0)),
        ],
        out_shape=[
            jax.ShapeDtypeStruct((1, _K), jnp.float32),
            jax.ShapeDtypeStruct((1, 1), jnp.float32),
        ],
    )(noise2d, gate11)


def _sc_body(x_hbm, n_hbm, src_hbm, nsrc_hbm, dst_hbm, off_hbm, sv_hbm,
             out_hbm, xbuf, nbuf, srcv, nsrcv, dstv, offv, svv):
    rows = x_hbm.shape[0] // _D
    rpw = rows // _NW
    wid = lax.axis_index("s") * _NC + lax.axis_index("c")
    base = wid * rpw

    pltpu.sync_copy(src_hbm, srcv)
    pltpu.sync_copy(nsrc_hbm, nsrcv)
    pltpu.sync_copy(dst_hbm, dstv)
    pltpu.sync_copy(off_hbm, offv)
    pltpu.sync_copy(sv_hbm, svv)

    lanes = lax.iota(jnp.int32, _L)
    masks = [lanes < (_K - _L * v) for v in range(_KP // _L)]
    src_r = [srcv[pl.ds(_L * v, _L)] for v in range(_KP // _L)]
    nsrc_r = [nsrcv[pl.ds(_L * v, _L)] for v in range(_KP // _L)]
    dst_r = [dstv[pl.ds(_L * v, _L)] for v in range(_KP // _L)]
    off_r = [offv[pl.ds(_L * v, _L)] for v in range(_KP // _L)]
    sv_r = svv[...]

    def chunk_body(ci, carry):
        r0 = base + ci * _CHUNK
        pltpu.sync_copy(x_hbm.at[pl.ds(r0 * _D, _CHUNK * _D)], xbuf)
        pltpu.sync_copy(n_hbm.at[pl.ds(r0 * _K, _CHUNK * _K)], nbuf)

        def row_body(r, rcarry):
            xb = jnp.full((_L,), r * _D, jnp.int32)
            nb = jnp.full((_L,), r * _K, jnp.int32)
            vals = []
            for v in range(_KP // _L):
                xg = plsc.load_gather(xbuf, [xb + src_r[v]], mask=masks[v])
                ng = plsc.load_gather(nbuf, [nb + nsrc_r[v]], mask=masks[v])
                vals.append(xg + sv_r * ng - off_r[v])
            for v in range(_KP // _L):
                plsc.store_scatter(xbuf, [xb + dst_r[v]], vals[v],
                                   mask=masks[v])
            return rcarry

        lax.fori_loop(0, _CHUNK, row_body, 0)
        pltpu.sync_copy(xbuf, out_hbm.at[pl.ds(r0 * _D, _CHUNK * _D)])
        return carry

    lax.fori_loop(0, rpw // _CHUNK, chunk_body, 0)


def kernel(x, gate, noise, idx, perm):
    b, t, d = x.shape
    rows = b * t
    x2 = x.reshape(rows, d)
    n2 = noise.reshape(rows, _K)

    nsum, g11 = _noise_stats(n2, gate.reshape(1, 1), rows)
    g = g11.reshape(())
    mu2 = nsum[0] * (_JITTER / rows)

    pad = _KP - _K
    src = jnp.pad(idx[perm], (0, pad)).astype(jnp.int32)
    nsrc = jnp.pad(perm, (0, pad)).astype(jnp.int32)
    dst = jnp.pad(idx, (0, pad)).astype(jnp.int32)
    off = jnp.pad(g * mu2[perm], (0, pad)).astype(jnp.float32)
    sv = jnp.full((_L,), _JITTER * g, jnp.float32)

    mesh = plsc.VectorSubcoreMesh(core_axis_name="c", subcore_axis_name="s",
                                  num_cores=_NC, num_subcores=_NS)
    out2 = pl.kernel(
        _sc_body,
        out_type=jax.ShapeDtypeStruct((rows * d,), jnp.float32),
        mesh=mesh,
        compiler_params=pltpu.CompilerParams(use_tc_tiling_on_sc=False,
                                             needs_layout_passes=False),
        scratch_types=[
            pltpu.VMEM((_CHUNK * d,), jnp.float32),
            pltpu.VMEM((_CHUNK * _K,), jnp.float32),
            pltpu.VMEM((_KP,), jnp.int32),
            pltpu.VMEM((_KP,), jnp.int32),
            pltpu.VMEM((_KP,), jnp.int32),
            pltpu.VMEM((_KP,), jnp.float32),
            pltpu.VMEM((_L,), jnp.float32),
        ],
    )(x2.reshape(-1), n2.reshape(-1), src, nsrc, dst, off, sv)

    return out2.reshape(b, t, d), g


# speedup vs baseline: 1.1109x; 1.1109x over previous
import functools

import jax
import jax.numpy as jnp
from jax import lax
from jax.experimental import pallas as pl
from jax.experimental.pallas import tpu as pltpu
from jax.experimental.pallas import tpu_sc as plsc

_D = 2048
_K = 36
_KP = 48
_JITTER = 0.02
_NC = 2
_NS = 16
_NW = _NC * _NS
_L = 16
_CHUNK = 16


def _stats_body(n_ref, gate_ref, sum_ref, g_ref):
    i = pl.program_id(0)
    s = jnp.sum(n_ref[...], axis=0, keepdims=True)

    @pl.when(i == 0)
    def _init():
        sum_ref[...] = s
        g_ref[...] = jax.nn.sigmoid(gate_ref[...])

    @pl.when(i != 0)
    def _acc():
        sum_ref[...] += s


def _noise_stats(noise2d, gate11, rows):
    blk = 1024
    grid = rows // blk
    return pl.pallas_call(
        _stats_body,
        grid=(grid,),
        in_specs=[
            pl.BlockSpec((blk, _K), lambda i: (i, 0)),
            pl.BlockSpec((1, 1), lambda i: (0, 0)),
        ],
        out_specs=[
            pl.BlockSpec((1, _K), lambda i: (0, 0)),
            pl.BlockSpec((1, 1), lambda i: (0, 0)),
        ],
        out_shape=[
            jax.ShapeDtypeStruct((1, _K), jnp.float32),
            jax.ShapeDtypeStruct((1, 1), jnp.float32),
        ],
    )(noise2d, gate11)


def _sc_body(x_hbm, n_hbm, src_hbm, nsrc_hbm, dst_hbm, off_hbm, sv_hbm,
             out_hbm, xbufs, nbufs, srcv, nsrcv, dstv, offv, svv,
             xin_sems, nin_sems, out_sems):
    rows = x_hbm.shape[0] // _D
    rpw = rows // _NW
    wid = lax.axis_index("s") * _NC + lax.axis_index("c")
    base = wid * rpw
    pairs = rpw // (2 * _CHUNK)

    pltpu.sync_copy(src_hbm, srcv)
    pltpu.sync_copy(nsrc_hbm, nsrcv)
    pltpu.sync_copy(dst_hbm, dstv)
    pltpu.sync_copy(off_hbm, offv)
    pltpu.sync_copy(sv_hbm, svv)

    lanes = lax.iota(jnp.int32, _L)
    masks = [lanes < (_K - _L * v) for v in range(_KP // _L)]
    src_r = [srcv[pl.ds(_L * v, _L)] for v in range(_KP // _L)]
    nsrc_r = [nsrcv[pl.ds(_L * v, _L)] for v in range(_KP // _L)]
    dst_r = [dstv[pl.ds(_L * v, _L)] for v in range(_KP // _L)]
    off_r = [offv[pl.ds(_L * v, _L)] for v in range(_KP // _L)]
    sv_r = svv[...]

    def start_in(b, ci):
        r0 = base + ci * _CHUNK
        pltpu.async_copy(x_hbm.at[pl.ds(r0 * _D, _CHUNK * _D)], xbufs[b],
                         xin_sems[b])
        pltpu.async_copy(n_hbm.at[pl.ds(r0 * _K, _CHUNK * _K)], nbufs[b],
                         nin_sems[b])

    def wait_in(b):
        pltpu.make_async_copy(x_hbm.at[pl.ds(0, _CHUNK * _D)], xbufs[b],
                              xin_sems[b]).wait()
        pltpu.make_async_copy(n_hbm.at[pl.ds(0, _CHUNK * _K)], nbufs[b],
                              nin_sems[b]).wait()

    def start_out(b, ci):
        r0 = base + ci * _CHUNK
        pltpu.async_copy(xbufs[b], out_hbm.at[pl.ds(r0 * _D, _CHUNK * _D)],
                         out_sems[b])

    def wait_out(b):
        pltpu.make_async_copy(xbufs[b], out_hbm.at[pl.ds(0, _CHUNK * _D)],
                              out_sems[b]).wait()

    def compute(b):
        def row_body(r, rcarry):
            xb = jnp.full((_L,), r * _D, jnp.int32)
            nb = jnp.full((_L,), r * _K, jnp.int32)
            vals = []
            for v in range(_KP // _L):
                xg = plsc.load_gather(xbufs[b], [xb + src_r[v]],
                                      mask=masks[v])
                ng = plsc.load_gather(nbufs[b], [nb + nsrc_r[v]],
                                      mask=masks[v])
                vals.append(xg + sv_r * ng - off_r[v])
            for v in range(_KP // _L):
                plsc.store_scatter(xbufs[b], [xb + dst_r[v]], vals[v],
                                   mask=masks[v])
            return rcarry

        lax.fori_loop(0, _CHUNK, row_body, 0)

    start_in(0, 0)

    def pair_body(i, carry):
        @pl.when(i > 0)
        def _():
            wait_out(1)

        start_in(1, 2 * i + 1)
        wait_in(0)
        compute(0)
        start_out(0, 2 * i)
        wait_in(1)
        compute(1)
        start_out(1, 2 * i + 1)

        @pl.when(i < pairs - 1)
        def _():
            wait_out(0)
            start_in(0, 2 * i + 2)

        return carry

    lax.fori_loop(0, pairs, pair_body, 0)
    wait_out(0)
    wait_out(1)


def kernel(x, gate, noise, idx, perm):
    b, t, d = x.shape
    rows = b * t
    x2 = x.reshape(rows, d)
    n2 = noise.reshape(rows, _K)

    nsum, g11 = _noise_stats(n2, gate.reshape(1, 1), rows)
    g = g11.reshape(())
    mu2 = nsum[0] * (_JITTER / rows)

    pad = _KP - _K
    src = jnp.pad(idx[perm], (0, pad)).astype(jnp.int32)
    nsrc = jnp.pad(perm, (0, pad)).astype(jnp.int32)
    dst = jnp.pad(idx, (0, pad)).astype(jnp.int32)
    off = jnp.pad(g * mu2[perm], (0, pad)).astype(jnp.float32)
    sv = jnp.full((_L,), _JITTER * g, jnp.float32)

    mesh = plsc.VectorSubcoreMesh(core_axis_name="c", subcore_axis_name="s",
                                  num_cores=_NC, num_subcores=_NS)
    out2 = pl.kernel(
        _sc_body,
        out_type=jax.ShapeDtypeStruct((rows * d,), jnp.float32),
        mesh=mesh,
        compiler_params=pltpu.CompilerParams(use_tc_tiling_on_sc=False,
                                             needs_layout_passes=False),
        scratch_types=[
            [pltpu.VMEM((_CHUNK * d,), jnp.float32) for _ in range(2)],
            [pltpu.VMEM((_CHUNK * _K,), jnp.float32) for _ in range(2)],
            pltpu.VMEM((_KP,), jnp.int32),
            pltpu.VMEM((_KP,), jnp.int32),
            pltpu.VMEM((_KP,), jnp.int32),
            pltpu.VMEM((_KP,), jnp.float32),
            pltpu.VMEM((_L,), jnp.float32),
            [pltpu.SemaphoreType.DMA for _ in range(2)],
            [pltpu.SemaphoreType.DMA for _ in range(2)],
            [pltpu.SemaphoreType.DMA for _ in range(2)],
        ],
    )(x2.reshape(-1), n2.reshape(-1), src, nsrc, dst, off, sv)

    return out2.reshape(b, t, d), g


# speedup vs baseline: 2.5417x; 2.2879x over previous
import functools

import jax
import jax.numpy as jnp
from jax import lax
from jax.experimental import pallas as pl
from jax.experimental.pallas import tpu as pltpu
from jax.experimental.pallas import tpu_sc as plsc

_D = 2048
_K = 36
_KP = 48
_JITTER = 0.02
_NC = 2
_NS = 16
_NW = _NC * _NS
_L = 16
_CHUNK = 16


def _stats_body(n_ref, gate_ref, sum_ref, g_ref):
    i = pl.program_id(0)
    s = jnp.sum(n_ref[...], axis=0, keepdims=True)

    @pl.when(i == 0)
    def _init():
        sum_ref[...] = s
        g_ref[...] = jax.nn.sigmoid(gate_ref[...])

    @pl.when(i != 0)
    def _acc():
        sum_ref[...] += s


def _noise_stats(noise2d, gate11, rows):
    blk = 1024
    grid = rows // blk
    return pl.pallas_call(
        _stats_body,
        grid=(grid,),
        in_specs=[
            pl.BlockSpec((blk, _K), lambda i: (i, 0)),
            pl.BlockSpec((1, 1), lambda i: (0, 0)),
        ],
        out_specs=[
            pl.BlockSpec((1, _K), lambda i: (0, 0)),
            pl.BlockSpec((1, 1), lambda i: (0, 0)),
        ],
        out_shape=[
            jax.ShapeDtypeStruct((1, _K), jnp.float32),
            jax.ShapeDtypeStruct((1, 1), jnp.float32),
        ],
    )(noise2d, gate11)


def _sc_body(x_hbm, n_hbm, src_hbm, nsrc_hbm, dst_hbm, off_hbm, sv_hbm,
             out_hbm, xbufs, nbufs, srcv, nsrcv, dstv, offv, svv,
             xin_sems, nin_sems, out_sems):
    rows = x_hbm.shape[0]
    rpw = rows // _NW
    wid = lax.axis_index("s") * _NC + lax.axis_index("c")
    base = wid * rpw
    pairs = rpw // (2 * _CHUNK)

    pltpu.sync_copy(src_hbm, srcv)
    pltpu.sync_copy(nsrc_hbm, nsrcv)
    pltpu.sync_copy(dst_hbm, dstv)
    pltpu.sync_copy(off_hbm, offv)
    pltpu.sync_copy(sv_hbm, svv)

    lanes = lax.iota(jnp.int32, _L)
    masks = [lanes < (_K - _L * v) for v in range(_KP // _L)]
    src_r = [srcv[pl.ds(_L * v, _L)] for v in range(_KP // _L)]
    nsrc_r = [nsrcv[pl.ds(_L * v, _L)] for v in range(_KP // _L)]
    dst_r = [dstv[pl.ds(_L * v, _L)] for v in range(_KP // _L)]
    off_r = [offv[pl.ds(_L * v, _L)] for v in range(_KP // _L)]
    sv_r = svv[...]

    def start_in(b, ci):
        r0 = base + ci * _CHUNK
        pltpu.async_copy(x_hbm.at[pl.ds(r0, _CHUNK)], xbufs[b],
                         xin_sems[b])
        pltpu.async_copy(n_hbm.at[pl.ds(r0 * _K, _CHUNK * _K)], nbufs[b],
                         nin_sems[b])

    def wait_in(b):
        pltpu.make_async_copy(x_hbm.at[pl.ds(0, _CHUNK)], xbufs[b],
                              xin_sems[b]).wait()
        pltpu.make_async_copy(n_hbm.at[pl.ds(0, _CHUNK * _K)], nbufs[b],
                              nin_sems[b]).wait()

    def start_out(b, ci):
        r0 = base + ci * _CHUNK
        pltpu.async_copy(xbufs[b], out_hbm.at[pl.ds(r0, _CHUNK)],
                         out_sems[b])

    def wait_out(b):
        pltpu.make_async_copy(xbufs[b], out_hbm.at[pl.ds(0, _CHUNK)],
                              out_sems[b]).wait()

    def compute(b):
        def row_body(r, rcarry):
            rv = jnp.full((_L,), r, jnp.int32)
            nb = jnp.full((_L,), r * _K, jnp.int32)
            vals = []
            for v in range(_KP // _L):
                xg = plsc.load_gather(xbufs[b], [rv, src_r[v]],
                                      mask=masks[v])
                ng = plsc.load_gather(nbufs[b], [nb + nsrc_r[v]],
                                      mask=masks[v])
                vals.append(xg + sv_r * ng - off_r[v])
            for v in range(_KP // _L):
                plsc.store_scatter(xbufs[b], [rv, dst_r[v]], vals[v],
                                   mask=masks[v])
            return rcarry

        lax.fori_loop(0, _CHUNK, row_body, 0)

    start_in(0, 0)

    def pair_body(i, carry):
        @pl.when(i > 0)
        def _():
            wait_out(1)

        start_in(1, 2 * i + 1)
        wait_in(0)
        compute(0)
        start_out(0, 2 * i)
        wait_in(1)
        compute(1)
        start_out(1, 2 * i + 1)

        @pl.when(i < pairs - 1)
        def _():
            wait_out(0)
            start_in(0, 2 * i + 2)

        return carry

    lax.fori_loop(0, pairs, pair_body, 0)
    wait_out(0)
    wait_out(1)


def kernel(x, gate, noise, idx, perm):
    b, t, d = x.shape
    rows = b * t
    x2 = x.reshape(rows, d)
    n2 = noise.reshape(rows, _K)

    nsum, g11 = _noise_stats(n2, gate.reshape(1, 1), rows)
    g = g11.reshape(())
    mu2 = nsum[0] * (_JITTER / rows)

    pad = _KP - _K
    src = jnp.pad(idx[perm], (0, pad)).astype(jnp.int32)
    nsrc = jnp.pad(perm, (0, pad)).astype(jnp.int32)
    dst = jnp.pad(idx, (0, pad)).astype(jnp.int32)
    off = jnp.pad(g * mu2[perm], (0, pad)).astype(jnp.float32)
    sv = jnp.full((_L,), _JITTER * g, jnp.float32)

    mesh = plsc.VectorSubcoreMesh(core_axis_name="c", subcore_axis_name="s",
                                  num_cores=_NC, num_subcores=_NS)
    out2 = pl.kernel(
        _sc_body,
        out_type=jax.ShapeDtypeStruct((rows, d), jnp.float32),
        mesh=mesh,
        compiler_params=pltpu.CompilerParams(needs_layout_passes=False),
        scratch_types=[
            [pltpu.VMEM((_CHUNK, d), jnp.float32) for _ in range(2)],
            [pltpu.VMEM((_CHUNK * _K,), jnp.float32) for _ in range(2)],
            pltpu.VMEM((_KP,), jnp.int32),
            pltpu.VMEM((_KP,), jnp.int32),
            pltpu.VMEM((_KP,), jnp.int32),
            pltpu.VMEM((_KP,), jnp.float32),
            pltpu.VMEM((_L,), jnp.float32),
            [pltpu.SemaphoreType.DMA for _ in range(2)],
            [pltpu.SemaphoreType.DMA for _ in range(2)],
            [pltpu.SemaphoreType.DMA for _ in range(2)],
        ],
    )(x2, n2.reshape(-1), src, nsrc, dst, off, sv)

    return out2.reshape(b, t, d), g


# speedup vs baseline: 2.7994x; 1.1014x over previous
import jax
import jax.numpy as jnp
from jax import lax
from jax.experimental import pallas as pl
from jax.experimental.pallas import tpu as pltpu
from jax.experimental.pallas import tpu_sc as plsc

_D = 2048
_K = 36
_KP = 48
_NV = _KP // 16
_JITTER = 0.02
_NC = 2
_NS = 16
_NW = _NC * _NS
_L = 16
_CHUNK = 16
_SB = 128


def _sc_body(x_hbm, n_hbm, idx_hbm, perm_hbm, gate_hbm,
             out_hbm, g_hbm,
             xbufs, nbufs, statsbuf, idxv, permv, meanv, gbuf, stage48,
             allv, sbuf, xin_sems, nin_sems, out_sems):
    rows = x_hbm.shape[0]
    rpw = rows // _NW
    cid = lax.axis_index("c")
    sid = lax.axis_index("s")
    wid = sid * _NC + cid
    base = wid * rpw

    lanes = lax.iota(jnp.int32, _L)
    zeros_i = jnp.zeros((_L,), jnp.int32)
    zeros_f = jnp.zeros((_L,), jnp.float32)
    masks = [lanes < (_K - _L * v) for v in range(_NV)]
    chan = [lanes + _L * v for v in range(_NV)]

    for v in range(_NV):
        idxv[pl.ds(_L * v, _L)] = zeros_i
        permv[pl.ds(_L * v, _L)] = zeros_i
    gbuf[...] = zeros_f
    pltpu.sync_copy(idx_hbm, idxv.at[pl.ds(0, _K)])
    pltpu.sync_copy(perm_hbm, permv.at[pl.ds(0, _K)])
    pltpu.sync_copy(gate_hbm, gbuf.at[pl.ds(0, 1)])

    gv = plsc.load_gather(gbuf, [zeros_i])
    g_sig = 1.0 / (1.0 + jnp.exp(-gv))
    sv_r = g_sig * _JITTER

    acc = [zeros_f for _ in range(_NV)]
    srow0 = sid * (rows // _NS)

    def stats_chunk(k, accs):
        pltpu.sync_copy(n_hbm.at[pl.ds(srow0 + k * _SB, _SB)], statsbuf)

        def stats_row(r, a):
            rv = jnp.full((_L,), r, jnp.int32)
            return tuple(
                a[v] + jnp.where(
                    masks[v],
                    plsc.load_gather(statsbuf, [rv, chan[v]],
                                     mask=masks[v]),
                    0.0)
                for v in range(_NV))

        return lax.fori_loop(0, _SB, stats_row, accs)

    acc = lax.fori_loop(0, rows // _NS // _SB, stats_chunk, tuple(acc))
    for v in range(_NV):
        stage48[pl.ds(_L * v, _L)] = acc[v]
    pltpu.sync_copy(stage48, sbuf.at[pl.ds(sid * _KP, _KP)])
    plsc.subcore_barrier()
    pltpu.sync_copy(sbuf, allv)
    scale = _JITTER / rows
    for v in range(_NV):
        tot = zeros_f
        for t in range(_NS):
            tot = tot + allv[pl.ds(t * _KP + _L * v, _L)]
        meanv[pl.ds(_L * v, _L)] = tot * scale

    nsrc_r = [permv[pl.ds(_L * v, _L)] for v in range(_NV)]
    dst_r = [idxv[pl.ds(_L * v, _L)] for v in range(_NV)]
    src_r = [plsc.load_gather(idxv, [nsrc_r[v]]) for v in range(_NV)]
    off_r = [g_sig * plsc.load_gather(meanv, [nsrc_r[v]])
             for v in range(_NV)]

    @pl.when(wid == 0)
    def _():
        gbuf[...] = g_sig
        pltpu.sync_copy(gbuf, g_hbm)

    pairs = rpw // (2 * _CHUNK)

    def start_in(b, ci):
        r0 = base + ci * _CHUNK
        pltpu.async_copy(x_hbm.at[pl.ds(r0, _CHUNK)], xbufs[b],
                         xin_sems[b])
        pltpu.async_copy(n_hbm.at[pl.ds(r0, _CHUNK)], nbufs[b],
                         nin_sems[b])

    def wait_in(b):
        pltpu.make_async_copy(x_hbm.at[pl.ds(0, _CHUNK)], xbufs[b],
                              xin_sems[b]).wait()
        pltpu.make_async_copy(n_hbm.at[pl.ds(0, _CHUNK)], nbufs[b],
                              nin_sems[b]).wait()

    def start_out(b, ci):
        r0 = base + ci * _CHUNK
        pltpu.async_copy(xbufs[b], out_hbm.at[pl.ds(r0, _CHUNK)],
                         out_sems[b])

    def wait_out(b):
        pltpu.make_async_copy(xbufs[b], out_hbm.at[pl.ds(0, _CHUNK)],
                              out_sems[b]).wait()

    def compute(b):
        def row_body(r, rcarry):
            rv = jnp.full((_L,), r, jnp.int32)
            vals = []
            for v in range(_NV):
                xg = plsc.load_gather(xbufs[b], [rv, src_r[v]],
                                      mask=masks[v])
                ng = plsc.load_gather(nbufs[b], [rv, nsrc_r[v]],
                                      mask=masks[v])
                vals.append(xg + sv_r * ng - off_r[v])
            for v in range(_NV):
                plsc.store_scatter(xbufs[b], [rv, dst_r[v]], vals[v],
                                   mask=masks[v])
            return rcarry

        lax.fori_loop(0, _CHUNK, row_body, 0)

    start_in(0, 0)

    def pair_body(i, carry):
        @pl.when(i > 0)
        def _():
            wait_out(1)

        start_in(1, 2 * i + 1)
        wait_in(0)
        compute(0)
        start_out(0, 2 * i)
        wait_in(1)
        compute(1)
        start_out(1, 2 * i + 1)

        @pl.when(i < pairs - 1)
        def _():
            wait_out(0)
            start_in(0, 2 * i + 2)

        return carry

    lax.fori_loop(0, pairs, pair_body, 0)
    wait_out(0)
    wait_out(1)


def kernel(x, gate, noise, idx, perm):
    b, t, d = x.shape
    rows = b * t
    x2 = x.reshape(rows, d)
    n2 = noise.reshape(rows, _K)

    mesh = plsc.VectorSubcoreMesh(core_axis_name="c", subcore_axis_name="s",
                                  num_cores=_NC, num_subcores=_NS)
    out2, gout = pl.kernel(
        _sc_body,
        out_type=[
            jax.ShapeDtypeStruct((rows, d), jnp.float32),
            jax.ShapeDtypeStruct((_L,), jnp.float32),
        ],
        mesh=mesh,
        compiler_params=pltpu.CompilerParams(needs_layout_passes=False),
        scratch_types=[
            [pltpu.VMEM((_CHUNK, d), jnp.float32) for _ in range(2)],
            [pltpu.VMEM((_CHUNK, _K), jnp.float32) for _ in range(2)],
            pltpu.VMEM((_SB, _K), jnp.float32),
            pltpu.VMEM((_KP,), jnp.int32),
            pltpu.VMEM((_KP,), jnp.int32),
            pltpu.VMEM((_KP,), jnp.float32),
            pltpu.VMEM((_L,), jnp.float32),
            pltpu.VMEM((_KP,), jnp.float32),
            pltpu.VMEM((_NS * _KP,), jnp.float32),
            pltpu.VMEM_SHARED((_NS * _KP,), jnp.float32),
            [pltpu.SemaphoreType.DMA for _ in range(2)],
            [pltpu.SemaphoreType.DMA for _ in range(2)],
            [pltpu.SemaphoreType.DMA for _ in range(2)],
        ],
    )(x2, n2, idx.astype(jnp.int32), perm.astype(jnp.int32),
      gate.reshape(1).astype(jnp.float32))

    return out2.reshape(b, t, d), gout[0]


# speedup vs baseline: 2.9125x; 1.0404x over previous
import jax
import jax.numpy as jnp
from jax import lax
from jax.experimental import pallas as pl
from jax.experimental.pallas import tpu as pltpu
from jax.experimental.pallas import tpu_sc as plsc

_D = 2048
_K = 36
_KP = 48
_NV = _KP // 16
_JITTER = 0.02
_NC = 2
_NS = 16
_NW = _NC * _NS
_L = 16
_CHUNK = 8
_NB = 4
_SB = 128


def _sc_body(x_hbm, n_hbm, idx_hbm, perm_hbm, gate_hbm,
             out_hbm, g_hbm,
             xbufs, nbufs, statsbuf, idxv, permv, meanv, gbuf, stage48,
             allv, sbuf, xin_sems, nin_sems, out_sems):
    rows = x_hbm.shape[0]
    rpw = rows // _NW
    cid = lax.axis_index("c")
    sid = lax.axis_index("s")
    wid = sid * _NC + cid
    base = wid * rpw

    lanes = lax.iota(jnp.int32, _L)
    zeros_i = jnp.zeros((_L,), jnp.int32)
    zeros_f = jnp.zeros((_L,), jnp.float32)
    masks = [lanes < (_K - _L * v) for v in range(_NV)]
    chan = [lanes + _L * v for v in range(_NV)]

    for v in range(_NV):
        idxv[pl.ds(_L * v, _L)] = zeros_i
        permv[pl.ds(_L * v, _L)] = zeros_i
    gbuf[...] = zeros_f
    pltpu.sync_copy(idx_hbm, idxv.at[pl.ds(0, _K)])
    pltpu.sync_copy(perm_hbm, permv.at[pl.ds(0, _K)])
    pltpu.sync_copy(gate_hbm, gbuf.at[pl.ds(0, 1)])

    gv = plsc.load_gather(gbuf, [zeros_i])
    g_sig = 1.0 / (1.0 + jnp.exp(-gv))
    sv_r = g_sig * _JITTER

    acc = [zeros_f for _ in range(_NV)]
    srow0 = sid * (rows // _NS)

    def stats_chunk(k, accs):
        pltpu.sync_copy(n_hbm.at[pl.ds(srow0 + k * _SB, _SB)], statsbuf)

        def stats_row(r, a):
            rv = jnp.full((_L,), r, jnp.int32)
            return tuple(
                a[v] + jnp.where(
                    masks[v],
                    plsc.load_gather(statsbuf, [rv, chan[v]],
                                     mask=masks[v]),
                    0.0)
                for v in range(_NV))

        return lax.fori_loop(0, _SB, stats_row, accs)

    acc = lax.fori_loop(0, rows // _NS // _SB, stats_chunk, tuple(acc))
    for v in range(_NV):
        stage48[pl.ds(_L * v, _L)] = acc[v]
    pltpu.sync_copy(stage48, sbuf.at[pl.ds(sid * _KP, _KP)])
    plsc.subcore_barrier()
    pltpu.sync_copy(sbuf, allv)
    scale = _JITTER / rows
    for v in range(_NV):
        tot = zeros_f
        for t in range(_NS):
            tot = tot + allv[pl.ds(t * _KP + _L * v, _L)]
        meanv[pl.ds(_L * v, _L)] = tot * scale

    nsrc_r = [permv[pl.ds(_L * v, _L)] for v in range(_NV)]
    dst_r = [idxv[pl.ds(_L * v, _L)] for v in range(_NV)]
    src_r = [plsc.load_gather(idxv, [nsrc_r[v]]) for v in range(_NV)]
    off_r = [g_sig * plsc.load_gather(meanv, [nsrc_r[v]])
             for v in range(_NV)]

    @pl.when(wid == 0)
    def _():
        gbuf[...] = g_sig
        pltpu.sync_copy(gbuf, g_hbm)

    nchunks = rpw // _CHUNK

    def start_in(b, ci):
        r0 = base + ci * _CHUNK
        pltpu.async_copy(x_hbm.at[pl.ds(r0, _CHUNK)], xbufs[b],
                         xin_sems[b])
        pltpu.async_copy(n_hbm.at[pl.ds(r0, _CHUNK)], nbufs[b],
                         nin_sems[b])

    def wait_in(b):
        pltpu.make_async_copy(x_hbm.at[pl.ds(0, _CHUNK)], xbufs[b],
                              xin_sems[b]).wait()
        pltpu.make_async_copy(n_hbm.at[pl.ds(0, _CHUNK)], nbufs[b],
                              nin_sems[b]).wait()

    def start_out(b, ci):
        r0 = base + ci * _CHUNK
        pltpu.async_copy(xbufs[b], out_hbm.at[pl.ds(r0, _CHUNK)],
                         out_sems[b])

    def wait_out(b):
        pltpu.make_async_copy(xbufs[b], out_hbm.at[pl.ds(0, _CHUNK)],
                              out_sems[b]).wait()

    def compute(b):
        def row_body(r, rcarry):
            rv = jnp.full((_L,), r, jnp.int32)
            vals = []
            for v in range(_NV):
                xg = plsc.load_gather(xbufs[b], [rv, src_r[v]],
                                      mask=masks[v])
                ng = plsc.load_gather(nbufs[b], [rv, nsrc_r[v]],
                                      mask=masks[v])
                vals.append(xg + sv_r * ng - off_r[v])
            for v in range(_NV):
                plsc.store_scatter(xbufs[b], [rv, dst_r[v]], vals[v],
                                   mask=masks[v])
            return rcarry

        lax.fori_loop(0, _CHUNK, row_body, 0)

    start_in(0, 0)
    start_in(1, 1)

    def group_body(gi, carry):
        for b in range(_NB):
            ci = gi * _NB + b
            wait_in(b)
            compute(b)
            start_out(b, ci)
            b2 = (b + 2) % _NB

            @pl.when(ci >= 2)
            def _():
                wait_out(b2)

            @pl.when(ci + 2 < nchunks)
            def _():
                start_in(b2, ci + 2)

        return carry

    lax.fori_loop(0, nchunks // _NB, group_body, 0)
    wait_out((nchunks - 2) % _NB)
    wait_out((nchunks - 1) % _NB)


def kernel(x, gate, noise, idx, perm):
    b, t, d = x.shape
    rows = b * t
    x2 = x.reshape(rows, d)
    n2 = noise.reshape(rows, _K)

    mesh = plsc.VectorSubcoreMesh(core_axis_name="c", subcore_axis_name="s",
                                  num_cores=_NC, num_subcores=_NS)
    out2, gout = pl.kernel(
        _sc_body,
        out_type=[
            jax.ShapeDtypeStruct((rows, d), jnp.float32),
            jax.ShapeDtypeStruct((_L,), jnp.float32),
        ],
        mesh=mesh,
        compiler_params=pltpu.CompilerParams(needs_layout_passes=False),
        scratch_types=[
            [pltpu.VMEM((_CHUNK, d), jnp.float32) for _ in range(_NB)],
            [pltpu.VMEM((_CHUNK, _K), jnp.float32) for _ in range(_NB)],
            pltpu.VMEM((_SB, _K), jnp.float32),
            pltpu.VMEM((_KP,), jnp.int32),
            pltpu.VMEM((_KP,), jnp.int32),
            pltpu.VMEM((_KP,), jnp.float32),
            pltpu.VMEM((_L,), jnp.float32),
            pltpu.VMEM((_KP,), jnp.float32),
            pltpu.VMEM((_NS * _KP,), jnp.float32),
            pltpu.VMEM_SHARED((_NS * _KP,), jnp.float32),
            [pltpu.SemaphoreType.DMA for _ in range(_NB)],
            [pltpu.SemaphoreType.DMA for _ in range(_NB)],
            [pltpu.SemaphoreType.DMA for _ in range(_NB)],
        ],
    )(x2, n2, idx.astype(jnp.int32), perm.astype(jnp.int32),
      gate.reshape(1).astype(jnp.float32))

    return out2.reshape(b, t, d), gout[0]


# speedup vs baseline: 2.9687x; 1.0193x over previous
import jax
import jax.numpy as jnp
from jax import lax
from jax.experimental import pallas as pl
from jax.experimental.pallas import tpu as pltpu
from jax.experimental.pallas import tpu_sc as plsc

_D = 2048
_K = 36
_KP = 48
_NV = _KP // 16
_JITTER = 0.02
_NC = 2
_NS = 16
_NW = _NC * _NS
_L = 16
_CHUNK = 8
_NB = 4
_SB = 256


def _sc_body(x_hbm, n_hbm, idx_hbm, perm_hbm, gate_hbm,
             out_hbm, g_hbm,
             xbufs, nbufs, statsbuf, idxv, permv, meanv, gbuf, stage48,
             allv, sbuf, xin_sems, nin_sems, out_sems):
    rows = x_hbm.shape[0]
    rpw = rows // _NW
    cid = lax.axis_index("c")
    sid = lax.axis_index("s")
    wid = sid * _NC + cid
    base = wid * rpw

    lanes = lax.iota(jnp.int32, _L)
    zeros_i = jnp.zeros((_L,), jnp.int32)
    zeros_f = jnp.zeros((_L,), jnp.float32)
    masks = [lanes < (_K - _L * v) for v in range(_NV)]

    def _prologue_in(b, ci):
        r0 = base + ci * _CHUNK
        pltpu.async_copy(x_hbm.at[pl.ds(r0, _CHUNK)], xbufs[b],
                         xin_sems[b])
        pltpu.async_copy(n_hbm.at[pl.ds(r0, _CHUNK)], nbufs[b],
                         nin_sems[b])

    for v in range(_NV):
        idxv[pl.ds(_L * v, _L)] = zeros_i
        permv[pl.ds(_L * v, _L)] = zeros_i
    gbuf[...] = zeros_f
    pltpu.sync_copy(idx_hbm, idxv.at[pl.ds(0, _K)])
    pltpu.sync_copy(perm_hbm, permv.at[pl.ds(0, _K)])
    pltpu.sync_copy(gate_hbm, gbuf.at[pl.ds(0, 1)])

    gv = plsc.load_gather(gbuf, [zeros_i])
    g_sig = 1.0 / (1.0 + jnp.exp(-gv))
    sv_r = g_sig * _JITTER

    _prologue_in(0, 0)
    _prologue_in(1, 1)

    acc = [zeros_f for _ in range(_NV)]
    srow0 = sid * (rows // _NS)

    def stats_chunk(k, accs):
        pltpu.sync_copy(n_hbm.at[pl.ds(srow0 + k * _SB, _SB)], statsbuf)

        def stats_row(r, a):
            rv = jnp.full((_L,), r, jnp.int32)
            tail = jnp.where(
                masks[2],
                plsc.load_gather(statsbuf, [rv, lanes + 2 * _L],
                                 mask=masks[2]),
                0.0)
            return (a[0] + statsbuf[r, pl.ds(0, _L)],
                    a[1] + statsbuf[r, pl.ds(_L, _L)],
                    a[2] + tail)

        return lax.fori_loop(0, _SB, stats_row, accs)

    acc = lax.fori_loop(0, rows // _NS // _SB, stats_chunk, tuple(acc))
    for v in range(_NV):
        stage48[pl.ds(_L * v, _L)] = acc[v]
    pltpu.sync_copy(stage48, sbuf.at[pl.ds(sid * _KP, _KP)])
    plsc.subcore_barrier()
    pltpu.sync_copy(sbuf, allv)
    scale = _JITTER / rows
    for v in range(_NV):
        tot = zeros_f
        for t in range(_NS):
            tot = tot + allv[pl.ds(t * _KP + _L * v, _L)]
        meanv[pl.ds(_L * v, _L)] = tot * scale

    nsrc_r = [permv[pl.ds(_L * v, _L)] for v in range(_NV)]
    dst_r = [idxv[pl.ds(_L * v, _L)] for v in range(_NV)]
    src_r = [plsc.load_gather(idxv, [nsrc_r[v]]) for v in range(_NV)]
    off_r = [g_sig * plsc.load_gather(meanv, [nsrc_r[v]])
             for v in range(_NV)]

    @pl.when(wid == 0)
    def _():
        gbuf[...] = g_sig
        pltpu.sync_copy(gbuf, g_hbm)

    nchunks = rpw // _CHUNK

    start_in = _prologue_in

    def wait_in(b):
        pltpu.make_async_copy(x_hbm.at[pl.ds(0, _CHUNK)], xbufs[b],
                              xin_sems[b]).wait()
        pltpu.make_async_copy(n_hbm.at[pl.ds(0, _CHUNK)], nbufs[b],
                              nin_sems[b]).wait()

    def start_out(b, ci):
        r0 = base + ci * _CHUNK
        pltpu.async_copy(xbufs[b], out_hbm.at[pl.ds(r0, _CHUNK)],
                         out_sems[b])

    def wait_out(b):
        pltpu.make_async_copy(xbufs[b], out_hbm.at[pl.ds(0, _CHUNK)],
                              out_sems[b]).wait()

    def compute(b):
        def row_body(r, rcarry):
            rv = jnp.full((_L,), r, jnp.int32)
            vals = []
            for v in range(_NV):
                xg = plsc.load_gather(xbufs[b], [rv, src_r[v]],
                                      mask=masks[v])
                ng = plsc.load_gather(nbufs[b], [rv, nsrc_r[v]],
                                      mask=masks[v])
                vals.append(xg + sv_r * ng - off_r[v])
            for v in range(_NV):
                plsc.store_scatter(xbufs[b], [rv, dst_r[v]], vals[v],
                                   mask=masks[v])
            return rcarry

        lax.fori_loop(0, _CHUNK, row_body, 0)

    def group_body(gi, carry):
        for b in range(_NB):
            ci = gi * _NB + b
            wait_in(b)
            compute(b)
            start_out(b, ci)
            b2 = (b + 2) % _NB

            @pl.when(ci >= 2)
            def _():
                wait_out(b2)

            @pl.when(ci + 2 < nchunks)
            def _():
                start_in(b2, ci + 2)

        return carry

    lax.fori_loop(0, nchunks // _NB, group_body, 0)
    wait_out((nchunks - 2) % _NB)
    wait_out((nchunks - 1) % _NB)


def kernel(x, gate, noise, idx, perm):
    b, t, d = x.shape
    rows = b * t
    x2 = x.reshape(rows, d)
    n2 = noise.reshape(rows, _K)

    mesh = plsc.VectorSubcoreMesh(core_axis_name="c", subcore_axis_name="s",
                                  num_cores=_NC, num_subcores=_NS)
    out2, gout = pl.kernel(
        _sc_body,
        out_type=[
            jax.ShapeDtypeStruct((rows, d), jnp.float32),
            jax.ShapeDtypeStruct((_L,), jnp.float32),
        ],
        mesh=mesh,
        compiler_params=pltpu.CompilerParams(needs_layout_passes=False),
        scratch_types=[
            [pltpu.VMEM((_CHUNK, d), jnp.float32) for _ in range(_NB)],
            [pltpu.VMEM((_CHUNK, _K), jnp.float32) for _ in range(_NB)],
            pltpu.VMEM((_SB, _K), jnp.float32),
            pltpu.VMEM((_KP,), jnp.int32),
            pltpu.VMEM((_KP,), jnp.int32),
            pltpu.VMEM((_KP,), jnp.float32),
            pltpu.VMEM((_L,), jnp.float32),
            pltpu.VMEM((_KP,), jnp.float32),
            pltpu.VMEM((_NS * _KP,), jnp.float32),
            pltpu.VMEM_SHARED((_NS * _KP,), jnp.float32),
            [pltpu.SemaphoreType.DMA for _ in range(_NB)],
            [pltpu.SemaphoreType.DMA for _ in range(_NB)],
            [pltpu.SemaphoreType.DMA for _ in range(_NB)],
        ],
    )(x2, n2, idx.astype(jnp.int32), perm.astype(jnp.int32),
      gate.reshape(1).astype(jnp.float32))

    return out2.reshape(b, t, d), gout[0]
